# baseline (speedup 1.0000x reference)
import functools

import jax
import jax.numpy as jnp
from jax import lax
from jax.experimental import pallas as pl
from jax.experimental.pallas import tpu as pltpu
from jax.experimental.pallas import tpu_sc as plsc

N, C, L = 1024, 80, 256
R = N * C
NUM_CORES = 2
NUM_SUBCORES = 16
W = NUM_CORES * NUM_SUBCORES
N_TC = 1024
R0 = N_TC * C
RPW = (R - R0) // W
CH = 64
CPAD = 128
ACC = 2 * CPAD
LANE = 16
BN = 8

assert N_TC % 64 == 0 and RPW % LANE == 0


def _sc_body(s_hbm, t_hbm, tgt_hbm, out_hbm,
             tgt_v, idx_v, sbuf0, sbuf1, tbuf0, tbuf1, rs_v, acc_v,
             sem_s0, sem_s1, sem_t0, sem_t1):
    wid = lax.axis_index("s") * NUM_CORES + lax.axis_index("c")
    base = R0 + wid * RPW
    lanes = lax.iota(jnp.int32, LANE)
    zero16 = jnp.zeros((LANE,), jnp.float32)

    pltpu.sync_copy(tgt_hbm.at[pl.ds(base, RPW)], tgt_v)

    def zbody(i, carry):
        acc_v[pl.ds(i * LANE, LANE)] = zero16
        return carry
    lax.fori_loop(0, ACC // LANE, zbody, 0)

    def cbody(i, wptr):
        tv = tgt_v[pl.ds(i * LANE, LANE)]
        m = tv > 0
        mi = m.astype(jnp.int32)
        pos = wptr + plsc.cumsum(mi) - 1
        rowid = base + i * LANE + lanes
        plsc.store_scatter(idx_v, [pos], rowid, mask=m)
        cls = lax.rem(rowid, jnp.int32(C))
        plsc.addupdate_scatter(acc_v, [cls + CPAD], m.astype(jnp.float32))
        return wptr + jnp.sum(mi)
    count = lax.fori_loop(0, RPW // LANE, cbody, jnp.int32(0))

    basevec = jnp.full((LANE,), base, jnp.int32)
    for t in range(CH // LANE):
        plsc.store_scatter(idx_v, [count + t * LANE + lanes], basevec)

    nchunks = (count + CH - 1) // CH
    sbufs = (sbuf0, sbuf1)
    tbufs = (tbuf0, tbuf1)
    sems_s = (sem_s0, sem_s1)
    sems_t = (sem_t0, sem_t1)

    def fire(g, b):
        idxsl = idx_v.at[pl.ds(g * CH, CH)]
        pltpu.async_copy(s_hbm.at[idxsl], sbufs[b], sems_s[b])
        pltpu.async_copy(t_hbm.at[idxsl], tbufs[b], sems_t[b])

    def drain(b):
        dummy = s_hbm.at[pl.ds(0, CH)]
        pltpu.make_async_copy(dummy, sbufs[b], sems_s[b]).wait()
        pltpu.make_async_copy(dummy, tbufs[b], sems_t[b]).wait()

    def compute_chunk(g, b):
        sbuf = sbufs[b]
        tbuf = tbufs[b]

        def group_body(gg, carry):
            off = g * CH + gg * LANE
            rid16 = idx_v[pl.ds(off, LANE)]
            cls16 = lax.rem(rid16, jnp.int32(C))
            valid16 = (off + lanes) < count
            for j in range(LANE):
                rbase = gg * LANE + j
                accs = [zero16, zero16, zero16, zero16]
                for k in range(L // LANE):
                    sv = sbuf[rbase, pl.ds(k * LANE, LANE)]
                    tv2 = tbuf[rbase, pl.ds(k * LANE, LANE)]
                    d = sv - tv2
                    a = jnp.abs(d)
                    m = jnp.minimum(a, 1.0)
                    accs[k % 4] = accs[k % 4] + m * (a - 0.5 * m)
                rs_v[pl.ds(j * LANE, LANE)] = (accs[0] + accs[1]) + (accs[2] + accs[3])
            tots = [zero16, zero16, zero16, zero16]
            for col in range(LANE):
                tots[col % 4] = tots[col % 4] + plsc.load_gather(
                    rs_v, [lanes * LANE + col])
            tot = (tots[0] + tots[1]) + (tots[2] + tots[3])
            for j in range(LANE):
                mj = (lanes == j) & valid16
                plsc.addupdate_scatter(acc_v, [cls16], tot, mask=mj)
            return carry
        lax.fori_loop(0, CH // LANE, group_body, 0)

    @pl.when(nchunks > 0)
    def _():
        fire(0, 0)

    def pair_body(gp, carry):
        for b in (0, 1):
            g = gp * 2 + b

            @pl.when(g + 1 < nchunks)
            def _():
                fire(g + 1, 1 - b)

            @pl.when(g < nchunks)
            def _():
                drain(b)
                compute_chunk(g, b)
        return carry
    lax.fori_loop(0, (nchunks + 1) // 2, pair_body, 0)

    pltpu.sync_copy(acc_v, out_hbm.at[wid])


@functools.partial(
    pl.kernel,
    out_type=jax.ShapeDtypeStruct((W, ACC), jnp.float32),
    mesh=plsc.VectorSubcoreMesh(core_axis_name="c", subcore_axis_name="s"),
    compiler_params=pltpu.CompilerParams(needs_layout_passes=False),
    scratch_types=[
        pltpu.VMEM((RPW,), jnp.int32),
        pltpu.VMEM((RPW + CH,), jnp.int32),
        pltpu.VMEM((CH, L), jnp.float32),
        pltpu.VMEM((CH, L), jnp.float32),
        pltpu.VMEM((CH, L), jnp.float32),
        pltpu.VMEM((CH, L), jnp.float32),
        pltpu.VMEM((LANE * LANE,), jnp.float32),
        pltpu.VMEM((ACC,), jnp.float32),
        pltpu.SemaphoreType.DMA,
        pltpu.SemaphoreType.DMA,
        pltpu.SemaphoreType.DMA,
        pltpu.SemaphoreType.DMA,
    ],
)
def _sc_partial_sums(s_hbm, t_hbm, tgt_hbm, out_hbm, *rest):
    _sc_body(s_hbm, t_hbm, tgt_hbm, out_hbm, *rest)


BR = 640


def _tc_body(s_ref, t_ref, mk_ref, out_ref, acc_ref):
    i = pl.program_id(0)

    @pl.when(i == 0)
    def _():
        acc_ref[...] = jnp.zeros_like(acc_ref)

    d = (s_ref[...] - t_ref[...]) * mk_ref[...]
    a = jnp.abs(d)
    m = jnp.minimum(a, 1.0)
    acc_ref[...] += m * (a - 0.5 * m)

    @pl.when(i == N_TC * C // BR - 1)
    def _():
        acc3 = acc_ref[...].reshape(BR // C, C, L)
        out_ref[...] = jnp.sum(acc3, axis=(0, 2))[None, :]


def _fin_body(sc_ref, tc_ref, mk_ref, out_ref):
    p = sc_ref[...]
    tc_s = tc_ref[...]
    tc_n = jnp.sum(mk_ref[...][:N_TC, :], axis=0, keepdims=True)
    s80 = jnp.sum(p[:, :CPAD], axis=0, keepdims=True)[:, :C] + tc_s
    n80 = jnp.sum(p[:, CPAD:], axis=0, keepdims=True)[:, :C] + tc_n
    denom = jnp.maximum(n80 * jnp.float32(L), 1.0)
    valid = (n80 > 1.0).astype(jnp.float32)
    out_ref[0, 0] = jnp.sum(s80 / denom * valid)


def kernel(le_student, le_teacher, targets):
    s2 = le_student.reshape(R, L)
    t2 = le_teacher.reshape(R, L)
    sc_parts = jnp.zeros((W, ACC), jnp.float32)

    mk = targets.astype(jnp.float32)
    mkcol = mk.reshape(R, 1)
    tc_parts = pl.pallas_call(
        _tc_body,
        grid=(N_TC * C // BR,),
        in_specs=[
            pl.BlockSpec((BR, L), lambda i: (i, 0)),
            pl.BlockSpec((BR, L), lambda i: (i, 0)),
            pl.BlockSpec((BR, 1), lambda i: (i, 0)),
        ],
        out_specs=pl.BlockSpec((1, C), lambda i: (0, 0)),
        out_shape=jax.ShapeDtypeStruct((1, C), jnp.float32),
        scratch_shapes=[pltpu.VMEM((BR, L), jnp.float32)],
    )(s2, t2, mkcol)

    loss = pl.pallas_call(
        _fin_body,
        out_shape=jax.ShapeDtypeStruct((1, 1), jnp.float32),
        out_specs=pl.BlockSpec(memory_space=pltpu.SMEM),
    )(sc_parts, tc_parts, mk)
    return loss[0, 0]


# speedup vs baseline: 1.1894x; 1.1894x over previous
import functools

import jax
import jax.numpy as jnp
from jax import lax
from jax.experimental import pallas as pl
from jax.experimental.pallas import tpu as pltpu
from jax.experimental.pallas import tpu_sc as plsc

N, C, L = 1024, 80, 256
R = N * C
NUM_CORES = 2
NUM_SUBCORES = 16
W = NUM_CORES * NUM_SUBCORES
N_TC = 1024
R0 = N_TC * C
RPW = (R - R0) // W
CH = 64
CPAD = 128
ACC = 2 * CPAD
LANE = 16
BN = 8

assert N_TC % 64 == 0 and RPW % LANE == 0


def _sc_body(s_hbm, t_hbm, tgt_hbm, out_hbm,
             tgt_v, idx_v, sbuf0, sbuf1, tbuf0, tbuf1, rs_v, acc_v,
             sem_s0, sem_s1, sem_t0, sem_t1):
    wid = lax.axis_index("s") * NUM_CORES + lax.axis_index("c")
    base = R0 + wid * RPW
    lanes = lax.iota(jnp.int32, LANE)
    zero16 = jnp.zeros((LANE,), jnp.float32)

    pltpu.sync_copy(tgt_hbm.at[pl.ds(base, RPW)], tgt_v)

    def zbody(i, carry):
        acc_v[pl.ds(i * LANE, LANE)] = zero16
        return carry
    lax.fori_loop(0, ACC // LANE, zbody, 0)

    def cbody(i, wptr):
        tv = tgt_v[pl.ds(i * LANE, LANE)]
        m = tv > 0
        mi = m.astype(jnp.int32)
        pos = wptr + plsc.cumsum(mi) - 1
        rowid = base + i * LANE + lanes
        plsc.store_scatter(idx_v, [pos], rowid, mask=m)
        cls = lax.rem(rowid, jnp.int32(C))
        plsc.addupdate_scatter(acc_v, [cls + CPAD], m.astype(jnp.float32))
        return wptr + jnp.sum(mi)
    count = lax.fori_loop(0, RPW // LANE, cbody, jnp.int32(0))

    basevec = jnp.full((LANE,), base, jnp.int32)
    for t in range(CH // LANE):
        plsc.store_scatter(idx_v, [count + t * LANE + lanes], basevec)

    nchunks = (count + CH - 1) // CH
    sbufs = (sbuf0, sbuf1)
    tbufs = (tbuf0, tbuf1)
    sems_s = (sem_s0, sem_s1)
    sems_t = (sem_t0, sem_t1)

    def fire(g, b):
        idxsl = idx_v.at[pl.ds(g * CH, CH)]
        pltpu.async_copy(s_hbm.at[idxsl], sbufs[b], sems_s[b])
        pltpu.async_copy(t_hbm.at[idxsl], tbufs[b], sems_t[b])

    def drain(b):
        dummy = s_hbm.at[pl.ds(0, CH)]
        pltpu.make_async_copy(dummy, sbufs[b], sems_s[b]).wait()
        pltpu.make_async_copy(dummy, tbufs[b], sems_t[b]).wait()

    def compute_chunk(g, b):
        sbuf = sbufs[b]
        tbuf = tbufs[b]

        def group_body(gg, carry):
            off = g * CH + gg * LANE
            rid16 = idx_v[pl.ds(off, LANE)]
            cls16 = lax.rem(rid16, jnp.int32(C))
            valid16 = (off + lanes) < count
            for j in range(LANE):
                rbase = gg * LANE + j
                accs = [zero16, zero16, zero16, zero16]
                for k in range(L // LANE):
                    sv = sbuf[rbase, pl.ds(k * LANE, LANE)]
                    tv2 = tbuf[rbase, pl.ds(k * LANE, LANE)]
                    d = sv - tv2
                    a = jnp.abs(d)
                    m = jnp.minimum(a, 1.0)
                    accs[k % 4] = accs[k % 4] + m * (a - 0.5 * m)
                rs_v[pl.ds(j * LANE, LANE)] = (accs[0] + accs[1]) + (accs[2] + accs[3])
            tots = [zero16, zero16, zero16, zero16]
            for col in range(LANE):
                tots[col % 4] = tots[col % 4] + plsc.load_gather(
                    rs_v, [lanes * LANE + col])
            tot = (tots[0] + tots[1]) + (tots[2] + tots[3])
            for j in range(LANE):
                mj = (lanes == j) & valid16
                plsc.addupdate_scatter(acc_v, [cls16], tot, mask=mj)
            return carry
        lax.fori_loop(0, CH // LANE, group_body, 0)

    @pl.when(nchunks > 0)
    def _():
        fire(0, 0)

    def pair_body(gp, carry):
        for b in (0, 1):
            g = gp * 2 + b

            @pl.when(g + 1 < nchunks)
            def _():
                fire(g + 1, 1 - b)

            @pl.when(g < nchunks)
            def _():
                drain(b)
                compute_chunk(g, b)
        return carry
    lax.fori_loop(0, (nchunks + 1) // 2, pair_body, 0)

    pltpu.sync_copy(acc_v, out_hbm.at[wid])


@functools.partial(
    pl.kernel,
    out_type=jax.ShapeDtypeStruct((W, ACC), jnp.float32),
    mesh=plsc.VectorSubcoreMesh(core_axis_name="c", subcore_axis_name="s"),
    compiler_params=pltpu.CompilerParams(needs_layout_passes=False),
    scratch_types=[
        pltpu.VMEM((RPW,), jnp.int32),
        pltpu.VMEM((RPW + CH,), jnp.int32),
        pltpu.VMEM((CH, L), jnp.float32),
        pltpu.VMEM((CH, L), jnp.float32),
        pltpu.VMEM((CH, L), jnp.float32),
        pltpu.VMEM((CH, L), jnp.float32),
        pltpu.VMEM((LANE * LANE,), jnp.float32),
        pltpu.VMEM((ACC,), jnp.float32),
        pltpu.SemaphoreType.DMA,
        pltpu.SemaphoreType.DMA,
        pltpu.SemaphoreType.DMA,
        pltpu.SemaphoreType.DMA,
    ],
)
def _sc_partial_sums(s_hbm, t_hbm, tgt_hbm, out_hbm, *rest):
    _sc_body(s_hbm, t_hbm, tgt_hbm, out_hbm, *rest)


BR = 640
NSTEPS_ALL = R // BR


def _tc_body(s_ref, t_ref, mkt_ref, out_ref, acc_ref):
    i = pl.program_id(0)

    @pl.when(i == 0)
    def _():
        acc_ref[...] = jnp.zeros_like(acc_ref)

    onehot = (lax.broadcasted_iota(jnp.int32, (NSTEPS_ALL, 1), 0)
              == i).astype(jnp.float32)
    mkc = jnp.dot(mkt_ref[...], onehot,
                  preferred_element_type=jnp.float32)
    d = (s_ref[...] - t_ref[...]) * mkc
    a = jnp.abs(d)
    m = jnp.minimum(a, 1.0)
    h = m * (a - 0.5 * m)
    acc_ref[...] += jnp.sum(h.reshape(BR // C, C, L), axis=0)

    @pl.when(i == N_TC * C // BR - 1)
    def _():
        out_ref[...] = jnp.sum(acc_ref[...], axis=1)[None, :]


def _fin_body(sc_ref, tc_ref, mk_ref, out_ref):
    p = sc_ref[...]
    tc_s = tc_ref[...]
    tc_n = jnp.sum(mk_ref[...][:N_TC, :], axis=0, keepdims=True)
    s80 = jnp.sum(p[:, :CPAD], axis=0, keepdims=True)[:, :C] + tc_s
    n80 = jnp.sum(p[:, CPAD:], axis=0, keepdims=True)[:, :C] + tc_n
    denom = jnp.maximum(n80 * jnp.float32(L), 1.0)
    valid = (n80 > 1.0).astype(jnp.float32)
    out_ref[0, 0] = jnp.sum(s80 / denom * valid)


def kernel(le_student, le_teacher, targets):
    s2 = le_student.reshape(R, L)
    t2 = le_teacher.reshape(R, L)
    sc_parts = jnp.zeros((W, ACC), jnp.float32)

    mk = targets.astype(jnp.float32)
    mkt = mk.reshape(NSTEPS_ALL, BR).T
    tc_parts = pl.pallas_call(
        _tc_body,
        grid=(N_TC * C // BR,),
        in_specs=[
            pl.BlockSpec((BR, L), lambda i: (i, 0)),
            pl.BlockSpec((BR, L), lambda i: (i, 0)),
            pl.BlockSpec((BR, NSTEPS_ALL), lambda i: (0, 0)),
        ],
        out_specs=pl.BlockSpec((1, C), lambda i: (0, 0)),
        out_shape=jax.ShapeDtypeStruct((1, C), jnp.float32),
        scratch_shapes=[pltpu.VMEM((C, L), jnp.float32)],
    )(s2, t2, mkt)

    loss = pl.pallas_call(
        _fin_body,
        out_shape=jax.ShapeDtypeStruct((1, 1), jnp.float32),
        out_specs=pl.BlockSpec(memory_space=pltpu.SMEM),
    )(sc_parts, tc_parts, mk)
    return loss[0, 0]


# speedup vs baseline: 1.6830x; 1.4150x over previous
import functools

import jax
import jax.numpy as jnp
from jax import lax
from jax.experimental import pallas as pl
from jax.experimental.pallas import tpu as pltpu
from jax.experimental.pallas import tpu_sc as plsc

N, C, L = 1024, 80, 256
R = N * C
NUM_CORES = 2
NUM_SUBCORES = 16
W = NUM_CORES * NUM_SUBCORES
N_TC = 1024
R0 = N_TC * C
RPW = (R - R0) // W
CH = 64
CPAD = 128
ACC = 2 * CPAD
LANE = 16
BN = 8

assert N_TC % 64 == 0 and RPW % LANE == 0


def _sc_body(s_hbm, t_hbm, tgt_hbm, out_hbm,
             tgt_v, idx_v, sbuf0, sbuf1, tbuf0, tbuf1, rs_v, acc_v,
             sem_s0, sem_s1, sem_t0, sem_t1):
    wid = lax.axis_index("s") * NUM_CORES + lax.axis_index("c")
    base = R0 + wid * RPW
    lanes = lax.iota(jnp.int32, LANE)
    zero16 = jnp.zeros((LANE,), jnp.float32)

    pltpu.sync_copy(tgt_hbm.at[pl.ds(base, RPW)], tgt_v)

    def zbody(i, carry):
        acc_v[pl.ds(i * LANE, LANE)] = zero16
        return carry
    lax.fori_loop(0, ACC // LANE, zbody, 0)

    def cbody(i, wptr):
        tv = tgt_v[pl.ds(i * LANE, LANE)]
        m = tv > 0
        mi = m.astype(jnp.int32)
        pos = wptr + plsc.cumsum(mi) - 1
        rowid = base + i * LANE + lanes
        plsc.store_scatter(idx_v, [pos], rowid, mask=m)
        cls = lax.rem(rowid, jnp.int32(C))
        plsc.addupdate_scatter(acc_v, [cls + CPAD], m.astype(jnp.float32))
        return wptr + jnp.sum(mi)
    count = lax.fori_loop(0, RPW // LANE, cbody, jnp.int32(0))

    basevec = jnp.full((LANE,), base, jnp.int32)
    for t in range(CH // LANE):
        plsc.store_scatter(idx_v, [count + t * LANE + lanes], basevec)

    nchunks = (count + CH - 1) // CH
    sbufs = (sbuf0, sbuf1)
    tbufs = (tbuf0, tbuf1)
    sems_s = (sem_s0, sem_s1)
    sems_t = (sem_t0, sem_t1)

    def fire(g, b):
        idxsl = idx_v.at[pl.ds(g * CH, CH)]
        pltpu.async_copy(s_hbm.at[idxsl], sbufs[b], sems_s[b])
        pltpu.async_copy(t_hbm.at[idxsl], tbufs[b], sems_t[b])

    def drain(b):
        dummy = s_hbm.at[pl.ds(0, CH)]
        pltpu.make_async_copy(dummy, sbufs[b], sems_s[b]).wait()
        pltpu.make_async_copy(dummy, tbufs[b], sems_t[b]).wait()

    def compute_chunk(g, b):
        sbuf = sbufs[b]
        tbuf = tbufs[b]

        def group_body(gg, carry):
            off = g * CH + gg * LANE
            rid16 = idx_v[pl.ds(off, LANE)]
            cls16 = lax.rem(rid16, jnp.int32(C))
            valid16 = (off + lanes) < count
            for j in range(LANE):
                rbase = gg * LANE + j
                accs = [zero16, zero16, zero16, zero16]
                for k in range(L // LANE):
                    sv = sbuf[rbase, pl.ds(k * LANE, LANE)]
                    tv2 = tbuf[rbase, pl.ds(k * LANE, LANE)]
                    d = sv - tv2
                    a = jnp.abs(d)
                    m = jnp.minimum(a, 1.0)
                    accs[k % 4] = accs[k % 4] + m * (a - 0.5 * m)
                rs_v[pl.ds(j * LANE, LANE)] = (accs[0] + accs[1]) + (accs[2] + accs[3])
            tots = [zero16, zero16, zero16, zero16]
            for col in range(LANE):
                tots[col % 4] = tots[col % 4] + plsc.load_gather(
                    rs_v, [lanes * LANE + col])
            tot = (tots[0] + tots[1]) + (tots[2] + tots[3])
            for j in range(LANE):
                mj = (lanes == j) & valid16
                plsc.addupdate_scatter(acc_v, [cls16], tot, mask=mj)
            return carry
        lax.fori_loop(0, CH // LANE, group_body, 0)

    @pl.when(nchunks > 0)
    def _():
        fire(0, 0)

    def pair_body(gp, carry):
        for b in (0, 1):
            g = gp * 2 + b

            @pl.when(g + 1 < nchunks)
            def _():
                fire(g + 1, 1 - b)

            @pl.when(g < nchunks)
            def _():
                drain(b)
                compute_chunk(g, b)
        return carry
    lax.fori_loop(0, (nchunks + 1) // 2, pair_body, 0)

    pltpu.sync_copy(acc_v, out_hbm.at[wid])


@functools.partial(
    pl.kernel,
    out_type=jax.ShapeDtypeStruct((W, ACC), jnp.float32),
    mesh=plsc.VectorSubcoreMesh(core_axis_name="c", subcore_axis_name="s"),
    compiler_params=pltpu.CompilerParams(needs_layout_passes=False),
    scratch_types=[
        pltpu.VMEM((RPW,), jnp.int32),
        pltpu.VMEM((RPW + CH,), jnp.int32),
        pltpu.VMEM((CH, L), jnp.float32),
        pltpu.VMEM((CH, L), jnp.float32),
        pltpu.VMEM((CH, L), jnp.float32),
        pltpu.VMEM((CH, L), jnp.float32),
        pltpu.VMEM((LANE * LANE,), jnp.float32),
        pltpu.VMEM((ACC,), jnp.float32),
        pltpu.SemaphoreType.DMA,
        pltpu.SemaphoreType.DMA,
        pltpu.SemaphoreType.DMA,
        pltpu.SemaphoreType.DMA,
    ],
)
def _sc_partial_sums(s_hbm, t_hbm, tgt_hbm, out_hbm, *rest):
    _sc_body(s_hbm, t_hbm, tgt_hbm, out_hbm, *rest)


BR = 1280
NSTEPS_ALL = R // BR


def _tc_body(s_ref, t_ref, mkt_ref, out_ref, acc_ref):
    i = pl.program_id(0)

    @pl.when(i == 0)
    def _():
        acc_ref[...] = jnp.zeros_like(acc_ref)

    onehot = (lax.broadcasted_iota(jnp.int32, (NSTEPS_ALL, 1), 0)
              == i).astype(jnp.float32)
    mkc = jnp.dot(mkt_ref[...], onehot,
                  preferred_element_type=jnp.float32)
    d = (s_ref[...] - t_ref[...]) * mkc
    a = jnp.abs(d)
    m = jnp.minimum(a, 1.0)
    h = m * (a - 0.5 * m)
    acc_ref[...] += jnp.sum(h.reshape(BR // C, C, L), axis=0)

    @pl.when(i == N_TC * C // BR - 1)
    def _():
        out_ref[...] = jnp.sum(acc_ref[...], axis=1)[None, :]


def _fin_body(sc_ref, tc_ref, mk_ref, out_ref):
    p = sc_ref[...]
    tc_s = tc_ref[...]
    tc_n = jnp.sum(mk_ref[...][:N_TC, :], axis=0, keepdims=True)
    s80 = jnp.sum(p[:, :CPAD], axis=0, keepdims=True)[:, :C] + tc_s
    n80 = jnp.sum(p[:, CPAD:], axis=0, keepdims=True)[:, :C] + tc_n
    denom = jnp.maximum(n80 * jnp.float32(L), 1.0)
    valid = (n80 > 1.0).astype(jnp.float32)
    out_ref[0, 0] = jnp.sum(s80 / denom * valid)


def kernel(le_student, le_teacher, targets):
    s2 = le_student.reshape(R, L)
    t2 = le_teacher.reshape(R, L)
    sc_parts = jnp.zeros((W, ACC), jnp.float32)

    mk = targets.astype(jnp.float32)
    mkt = mk.reshape(NSTEPS_ALL, BR).T
    tc_parts = pl.pallas_call(
        _tc_body,
        grid=(N_TC * C // BR,),
        in_specs=[
            pl.BlockSpec((BR, L), lambda i: (i, 0)),
            pl.BlockSpec((BR, L), lambda i: (i, 0)),
            pl.BlockSpec((BR, NSTEPS_ALL), lambda i: (0, 0)),
        ],
        out_specs=pl.BlockSpec((1, C), lambda i: (0, 0)),
        out_shape=jax.ShapeDtypeStruct((1, C), jnp.float32),
        scratch_shapes=[pltpu.VMEM((C, L), jnp.float32)],
    )(s2, t2, mkt)

    loss = pl.pallas_call(
        _fin_body,
        out_shape=jax.ShapeDtypeStruct((1, 1), jnp.float32),
        out_specs=pl.BlockSpec(memory_space=pltpu.SMEM),
    )(sc_parts, tc_parts, mk)
    return loss[0, 0]


# speedup vs baseline: 2.1867x; 1.2993x over previous
import functools

import jax
import jax.numpy as jnp
from jax import lax
from jax.experimental import pallas as pl
from jax.experimental.pallas import tpu as pltpu
from jax.experimental.pallas import tpu_sc as plsc

N, C, L = 1024, 80, 256
R = N * C
NUM_CORES = 2
NUM_SUBCORES = 16
W = NUM_CORES * NUM_SUBCORES
N_TC = 1024
R0 = N_TC * C
RPW = (R - R0) // W
CH = 64
CPAD = 128
ACC = 2 * CPAD
LANE = 16
BN = 8

assert N_TC % 64 == 0 and RPW % LANE == 0


def _sc_body(s_hbm, t_hbm, tgt_hbm, out_hbm,
             tgt_v, idx_v, sbuf0, sbuf1, tbuf0, tbuf1, rs_v, acc_v,
             sem_s0, sem_s1, sem_t0, sem_t1):
    wid = lax.axis_index("s") * NUM_CORES + lax.axis_index("c")
    base = R0 + wid * RPW
    lanes = lax.iota(jnp.int32, LANE)
    zero16 = jnp.zeros((LANE,), jnp.float32)

    pltpu.sync_copy(tgt_hbm.at[pl.ds(base, RPW)], tgt_v)

    def zbody(i, carry):
        acc_v[pl.ds(i * LANE, LANE)] = zero16
        return carry
    lax.fori_loop(0, ACC // LANE, zbody, 0)

    def cbody(i, wptr):
        tv = tgt_v[pl.ds(i * LANE, LANE)]
        m = tv > 0
        mi = m.astype(jnp.int32)
        pos = wptr + plsc.cumsum(mi) - 1
        rowid = base + i * LANE + lanes
        plsc.store_scatter(idx_v, [pos], rowid, mask=m)
        cls = lax.rem(rowid, jnp.int32(C))
        plsc.addupdate_scatter(acc_v, [cls + CPAD], m.astype(jnp.float32))
        return wptr + jnp.sum(mi)
    count = lax.fori_loop(0, RPW // LANE, cbody, jnp.int32(0))

    basevec = jnp.full((LANE,), base, jnp.int32)
    for t in range(CH // LANE):
        plsc.store_scatter(idx_v, [count + t * LANE + lanes], basevec)

    nchunks = (count + CH - 1) // CH
    sbufs = (sbuf0, sbuf1)
    tbufs = (tbuf0, tbuf1)
    sems_s = (sem_s0, sem_s1)
    sems_t = (sem_t0, sem_t1)

    def fire(g, b):
        idxsl = idx_v.at[pl.ds(g * CH, CH)]
        pltpu.async_copy(s_hbm.at[idxsl], sbufs[b], sems_s[b])
        pltpu.async_copy(t_hbm.at[idxsl], tbufs[b], sems_t[b])

    def drain(b):
        dummy = s_hbm.at[pl.ds(0, CH)]
        pltpu.make_async_copy(dummy, sbufs[b], sems_s[b]).wait()
        pltpu.make_async_copy(dummy, tbufs[b], sems_t[b]).wait()

    def compute_chunk(g, b):
        sbuf = sbufs[b]
        tbuf = tbufs[b]

        def group_body(gg, carry):
            off = g * CH + gg * LANE
            rid16 = idx_v[pl.ds(off, LANE)]
            cls16 = lax.rem(rid16, jnp.int32(C))
            valid16 = (off + lanes) < count
            for j in range(LANE):
                rbase = gg * LANE + j
                accs = [zero16, zero16, zero16, zero16]
                for k in range(L // LANE):
                    sv = sbuf[rbase, pl.ds(k * LANE, LANE)]
                    tv2 = tbuf[rbase, pl.ds(k * LANE, LANE)]
                    d = sv - tv2
                    a = jnp.abs(d)
                    m = jnp.minimum(a, 1.0)
                    accs[k % 4] = accs[k % 4] + m * (a - 0.5 * m)
                rs_v[pl.ds(j * LANE, LANE)] = (accs[0] + accs[1]) + (accs[2] + accs[3])
            tots = [zero16, zero16, zero16, zero16]
            for col in range(LANE):
                tots[col % 4] = tots[col % 4] + plsc.load_gather(
                    rs_v, [lanes * LANE + col])
            tot = (tots[0] + tots[1]) + (tots[2] + tots[3])
            for j in range(LANE):
                mj = (lanes == j) & valid16
                plsc.addupdate_scatter(acc_v, [cls16], tot, mask=mj)
            return carry
        lax.fori_loop(0, CH // LANE, group_body, 0)

    @pl.when(nchunks > 0)
    def _():
        fire(0, 0)

    def pair_body(gp, carry):
        for b in (0, 1):
            g = gp * 2 + b

            @pl.when(g + 1 < nchunks)
            def _():
                fire(g + 1, 1 - b)

            @pl.when(g < nchunks)
            def _():
                drain(b)
                compute_chunk(g, b)
        return carry
    lax.fori_loop(0, (nchunks + 1) // 2, pair_body, 0)

    pltpu.sync_copy(acc_v, out_hbm.at[wid])


@functools.partial(
    pl.kernel,
    out_type=jax.ShapeDtypeStruct((W, ACC), jnp.float32),
    mesh=plsc.VectorSubcoreMesh(core_axis_name="c", subcore_axis_name="s"),
    compiler_params=pltpu.CompilerParams(needs_layout_passes=False),
    scratch_types=[
        pltpu.VMEM((RPW,), jnp.int32),
        pltpu.VMEM((RPW + CH,), jnp.int32),
        pltpu.VMEM((CH, L), jnp.float32),
        pltpu.VMEM((CH, L), jnp.float32),
        pltpu.VMEM((CH, L), jnp.float32),
        pltpu.VMEM((CH, L), jnp.float32),
        pltpu.VMEM((LANE * LANE,), jnp.float32),
        pltpu.VMEM((ACC,), jnp.float32),
        pltpu.SemaphoreType.DMA,
        pltpu.SemaphoreType.DMA,
        pltpu.SemaphoreType.DMA,
        pltpu.SemaphoreType.DMA,
    ],
)
def _sc_partial_sums(s_hbm, t_hbm, tgt_hbm, out_hbm, *rest):
    _sc_body(s_hbm, t_hbm, tgt_hbm, out_hbm, *rest)


BR = 2560
NSTEPS_ALL = R // BR


def _tc_body(s_ref, t_ref, mkt_ref, out_ref, acc_ref):
    i = pl.program_id(0)

    @pl.when(i == 0)
    def _():
        acc_ref[...] = jnp.zeros_like(acc_ref)

    onehot = (lax.broadcasted_iota(jnp.int32, (NSTEPS_ALL, 1), 0)
              == i).astype(jnp.float32)
    mkc = jnp.dot(mkt_ref[...], onehot,
                  preferred_element_type=jnp.float32)
    d = (s_ref[...] - t_ref[...]) * mkc
    a = jnp.abs(d)
    m = jnp.minimum(a, 1.0)
    h = m * (a - 0.5 * m)
    acc_ref[...] += jnp.sum(h.reshape(BR // C, C, L), axis=0)

    @pl.when(i == N_TC * C // BR - 1)
    def _():
        out_ref[...] = jnp.sum(acc_ref[...], axis=1)[None, :]


def _fin_body(sc_ref, tc_ref, mk_ref, out_ref):
    p = sc_ref[...]
    tc_s = tc_ref[...]
    tc_n = jnp.sum(mk_ref[...][:N_TC, :], axis=0, keepdims=True)
    s80 = jnp.sum(p[:, :CPAD], axis=0, keepdims=True)[:, :C] + tc_s
    n80 = jnp.sum(p[:, CPAD:], axis=0, keepdims=True)[:, :C] + tc_n
    denom = jnp.maximum(n80 * jnp.float32(L), 1.0)
    valid = (n80 > 1.0).astype(jnp.float32)
    out_ref[0, 0] = jnp.sum(s80 / denom * valid)


def kernel(le_student, le_teacher, targets):
    s2 = le_student.reshape(R, L)
    t2 = le_teacher.reshape(R, L)
    sc_parts = jnp.zeros((W, ACC), jnp.float32)

    mk = targets.astype(jnp.float32)
    mkt = mk.reshape(NSTEPS_ALL, BR).T
    tc_parts = pl.pallas_call(
        _tc_body,
        grid=(N_TC * C // BR,),
        in_specs=[
            pl.BlockSpec((BR, L), lambda i: (i, 0)),
            pl.BlockSpec((BR, L), lambda i: (i, 0)),
            pl.BlockSpec((BR, NSTEPS_ALL), lambda i: (0, 0)),
        ],
        out_specs=pl.BlockSpec((1, C), lambda i: (0, 0)),
        out_shape=jax.ShapeDtypeStruct((1, C), jnp.float32),
        scratch_shapes=[pltpu.VMEM((C, L), jnp.float32)],
    )(s2, t2, mkt)

    loss = pl.pallas_call(
        _fin_body,
        out_shape=jax.ShapeDtypeStruct((1, 1), jnp.float32),
        out_specs=pl.BlockSpec(memory_space=pltpu.SMEM),
    )(sc_parts, tc_parts, mk)
    return loss[0, 0]


# speedup vs baseline: 2.4359x; 1.1140x over previous
import functools

import jax
import jax.numpy as jnp
from jax import lax
from jax.experimental import pallas as pl
from jax.experimental.pallas import tpu as pltpu
from jax.experimental.pallas import tpu_sc as plsc

N, C, L = 1024, 80, 256
R = N * C
NUM_CORES = 2
NUM_SUBCORES = 16
W = NUM_CORES * NUM_SUBCORES
N_TC = 1024
R0 = N_TC * C
RPW = (R - R0) // W
CH = 64
CPAD = 128
ACC = 2 * CPAD
LANE = 16
BN = 8

assert N_TC % 64 == 0 and RPW % LANE == 0


def _sc_body(s_hbm, t_hbm, tgt_hbm, out_hbm,
             tgt_v, idx_v, sbuf0, sbuf1, tbuf0, tbuf1, rs_v, acc_v,
             sem_s0, sem_s1, sem_t0, sem_t1):
    wid = lax.axis_index("s") * NUM_CORES + lax.axis_index("c")
    base = R0 + wid * RPW
    lanes = lax.iota(jnp.int32, LANE)
    zero16 = jnp.zeros((LANE,), jnp.float32)

    pltpu.sync_copy(tgt_hbm.at[pl.ds(base, RPW)], tgt_v)

    def zbody(i, carry):
        acc_v[pl.ds(i * LANE, LANE)] = zero16
        return carry
    lax.fori_loop(0, ACC // LANE, zbody, 0)

    def cbody(i, wptr):
        tv = tgt_v[pl.ds(i * LANE, LANE)]
        m = tv > 0
        mi = m.astype(jnp.int32)
        pos = wptr + plsc.cumsum(mi) - 1
        rowid = base + i * LANE + lanes
        plsc.store_scatter(idx_v, [pos], rowid, mask=m)
        cls = lax.rem(rowid, jnp.int32(C))
        plsc.addupdate_scatter(acc_v, [cls + CPAD], m.astype(jnp.float32))
        return wptr + jnp.sum(mi)
    count = lax.fori_loop(0, RPW // LANE, cbody, jnp.int32(0))

    basevec = jnp.full((LANE,), base, jnp.int32)
    for t in range(CH // LANE):
        plsc.store_scatter(idx_v, [count + t * LANE + lanes], basevec)

    nchunks = (count + CH - 1) // CH
    sbufs = (sbuf0, sbuf1)
    tbufs = (tbuf0, tbuf1)
    sems_s = (sem_s0, sem_s1)
    sems_t = (sem_t0, sem_t1)

    def fire(g, b):
        idxsl = idx_v.at[pl.ds(g * CH, CH)]
        pltpu.async_copy(s_hbm.at[idxsl], sbufs[b], sems_s[b])
        pltpu.async_copy(t_hbm.at[idxsl], tbufs[b], sems_t[b])

    def drain(b):
        dummy = s_hbm.at[pl.ds(0, CH)]
        pltpu.make_async_copy(dummy, sbufs[b], sems_s[b]).wait()
        pltpu.make_async_copy(dummy, tbufs[b], sems_t[b]).wait()

    def compute_chunk(g, b):
        sbuf = sbufs[b]
        tbuf = tbufs[b]

        def group_body(gg, carry):
            off = g * CH + gg * LANE
            rid16 = idx_v[pl.ds(off, LANE)]
            cls16 = lax.rem(rid16, jnp.int32(C))
            valid16 = (off + lanes) < count
            for j in range(LANE):
                rbase = gg * LANE + j
                accs = [zero16, zero16, zero16, zero16]
                for k in range(L // LANE):
                    sv = sbuf[rbase, pl.ds(k * LANE, LANE)]
                    tv2 = tbuf[rbase, pl.ds(k * LANE, LANE)]
                    d = sv - tv2
                    a = jnp.abs(d)
                    m = jnp.minimum(a, 1.0)
                    accs[k % 4] = accs[k % 4] + m * (a - 0.5 * m)
                rs_v[pl.ds(j * LANE, LANE)] = (accs[0] + accs[1]) + (accs[2] + accs[3])
            tots = [zero16, zero16, zero16, zero16]
            for col in range(LANE):
                tots[col % 4] = tots[col % 4] + plsc.load_gather(
                    rs_v, [lanes * LANE + col])
            tot = (tots[0] + tots[1]) + (tots[2] + tots[3])
            for j in range(LANE):
                mj = (lanes == j) & valid16
                plsc.addupdate_scatter(acc_v, [cls16], tot, mask=mj)
            return carry
        lax.fori_loop(0, CH // LANE, group_body, 0)

    @pl.when(nchunks > 0)
    def _():
        fire(0, 0)

    def pair_body(gp, carry):
        for b in (0, 1):
            g = gp * 2 + b

            @pl.when(g + 1 < nchunks)
            def _():
                fire(g + 1, 1 - b)

            @pl.when(g < nchunks)
            def _():
                drain(b)
                compute_chunk(g, b)
        return carry
    lax.fori_loop(0, (nchunks + 1) // 2, pair_body, 0)

    pltpu.sync_copy(acc_v, out_hbm.at[wid])


@functools.partial(
    pl.kernel,
    out_type=jax.ShapeDtypeStruct((W, ACC), jnp.float32),
    mesh=plsc.VectorSubcoreMesh(core_axis_name="c", subcore_axis_name="s"),
    compiler_params=pltpu.CompilerParams(needs_layout_passes=False),
    scratch_types=[
        pltpu.VMEM((RPW,), jnp.int32),
        pltpu.VMEM((RPW + CH,), jnp.int32),
        pltpu.VMEM((CH, L), jnp.float32),
        pltpu.VMEM((CH, L), jnp.float32),
        pltpu.VMEM((CH, L), jnp.float32),
        pltpu.VMEM((CH, L), jnp.float32),
        pltpu.VMEM((LANE * LANE,), jnp.float32),
        pltpu.VMEM((ACC,), jnp.float32),
        pltpu.SemaphoreType.DMA,
        pltpu.SemaphoreType.DMA,
        pltpu.SemaphoreType.DMA,
        pltpu.SemaphoreType.DMA,
    ],
)
def _sc_partial_sums(s_hbm, t_hbm, tgt_hbm, out_hbm, *rest):
    _sc_body(s_hbm, t_hbm, tgt_hbm, out_hbm, *rest)


BR = 5120
NSTEPS_ALL = R // BR


def _tc_body(s_ref, t_ref, mkt_ref, out_ref, acc_ref):
    i = pl.program_id(0)

    @pl.when(i == 0)
    def _():
        acc_ref[...] = jnp.zeros_like(acc_ref)

    onehot = (lax.broadcasted_iota(jnp.int32, (NSTEPS_ALL, 1), 0)
              == i).astype(jnp.float32)
    mkc = jnp.dot(mkt_ref[...], onehot,
                  preferred_element_type=jnp.float32)
    d = (s_ref[...] - t_ref[...]) * mkc
    a = jnp.abs(d)
    m = jnp.minimum(a, 1.0)
    h = m * (a - 0.5 * m)
    acc_ref[...] += jnp.sum(h.reshape(BR // C, C, L), axis=0)

    @pl.when(i == N_TC * C // BR - 1)
    def _():
        out_ref[...] = jnp.sum(acc_ref[...], axis=1)[None, :]


def _fin_body(sc_ref, tc_ref, mk_ref, out_ref):
    p = sc_ref[...]
    tc_s = tc_ref[...]
    tc_n = jnp.sum(mk_ref[...][:N_TC, :], axis=0, keepdims=True)
    s80 = jnp.sum(p[:, :CPAD], axis=0, keepdims=True)[:, :C] + tc_s
    n80 = jnp.sum(p[:, CPAD:], axis=0, keepdims=True)[:, :C] + tc_n
    denom = jnp.maximum(n80 * jnp.float32(L), 1.0)
    valid = (n80 > 1.0).astype(jnp.float32)
    out_ref[0, 0] = jnp.sum(s80 / denom * valid)


def kernel(le_student, le_teacher, targets):
    s2 = le_student.reshape(R, L)
    t2 = le_teacher.reshape(R, L)
    sc_parts = jnp.zeros((W, ACC), jnp.float32)

    mk = targets.astype(jnp.float32)
    mkt = mk.reshape(NSTEPS_ALL, BR).T
    tc_parts = pl.pallas_call(
        _tc_body,
        grid=(N_TC * C // BR,),
        in_specs=[
            pl.BlockSpec((BR, L), lambda i: (i, 0)),
            pl.BlockSpec((BR, L), lambda i: (i, 0)),
            pl.BlockSpec((BR, NSTEPS_ALL), lambda i: (0, 0)),
        ],
        out_specs=pl.BlockSpec((1, C), lambda i: (0, 0)),
        out_shape=jax.ShapeDtypeStruct((1, C), jnp.float32),
        scratch_shapes=[pltpu.VMEM((C, L), jnp.float32)],
    )(s2, t2, mkt)

    loss = pl.pallas_call(
        _fin_body,
        out_shape=jax.ShapeDtypeStruct((1, 1), jnp.float32),
        out_specs=pl.BlockSpec(memory_space=pltpu.SMEM),
    )(sc_parts, tc_parts, mk)
    return loss[0, 0]
